# baseline (speedup 1.0000x reference)
import jax
import jax.numpy as jnp
from jax.experimental import pallas as pl
from jax.experimental.pallas import tpu as pltpu

_TAIL_SPLIT = 7


def _body(vecs_ref, ids_ref, ones_ref, queue_ref, qids_ref, valid_ref,
          oq_ref, oids_ref, oval_ref, sem_ref):
    batch = vecs_ref.shape[0]
    k = queue_ref.shape[0]
    tail = k - batch
    chunk = tail // _TAIL_SPLIT

    copies = [
        pltpu.make_async_copy(vecs_ref, oq_ref.at[pl.ds(0, batch)], sem_ref.at[0]),
        pltpu.make_async_copy(ids_ref, oids_ref.at[pl.ds(0, batch)], sem_ref.at[1]),
        pltpu.make_async_copy(ones_ref, oval_ref.at[pl.ds(0, batch)], sem_ref.at[2]),
        pltpu.make_async_copy(qids_ref.at[pl.ds(batch, tail)],
                              oids_ref.at[pl.ds(batch, tail)], sem_ref.at[3]),
        pltpu.make_async_copy(valid_ref.at[pl.ds(batch, tail)],
                              oval_ref.at[pl.ds(batch, tail)], sem_ref.at[4]),
    ]
    for j in range(_TAIL_SPLIT):
        off = batch + j * chunk
        copies.append(pltpu.make_async_copy(
            queue_ref.at[pl.ds(off, chunk)],
            oq_ref.at[pl.ds(off, chunk)], sem_ref.at[5 + j]))
    for c in copies:
        c.start()
    for c in copies:
        c.wait()


def kernel(vecs, ids, queue, queue_ids, valid):
    batch, dim = vecs.shape
    k = queue.shape[0]
    ones = jnp.ones((batch,), dtype=jnp.int8)
    valid8 = valid.astype(jnp.int8)

    hbm = pl.BlockSpec(memory_space=pltpu.MemorySpace.HBM)
    oq, oids, oval8 = pl.pallas_call(
        _body,
        in_specs=[hbm] * 6,
        out_specs=[hbm] * 3,
        out_shape=[
            jax.ShapeDtypeStruct((k, dim), queue.dtype),
            jax.ShapeDtypeStruct((k,), queue_ids.dtype),
            jax.ShapeDtypeStruct((k,), jnp.int8),
        ],
        scratch_shapes=[pltpu.SemaphoreType.DMA((5 + _TAIL_SPLIT,))],
    )(vecs, ids, ones, queue, queue_ids, valid8)
    return (oq, oids, oval8.astype(jnp.bool_))


# speedup vs baseline: 42.8277x; 42.8277x over previous
import functools

import jax
import jax.numpy as jnp
from jax.experimental import pallas as pl

_LANES = 128
_BQ = 8192


def _body(vecs_ref, queue_ref, ids_ref, qids_ref, valid_ref,
          oq_ref, oids_ref, oval_ref, *, nvb):
    i = pl.program_id(0)

    @pl.when(i < nvb)
    def _():
        oq_ref[...] = vecs_ref[...]
        oids_ref[...] = ids_ref[...]
        oval_ref[...] = jnp.ones_like(oval_ref)

    @pl.when(i >= nvb)
    def _():
        oq_ref[...] = queue_ref[...]
        oids_ref[...] = qids_ref[...]
        oval_ref[...] = valid_ref[...]


def kernel(vecs, ids, queue, queue_ids, valid):
    batch, dim = vecs.shape
    k = queue.shape[0]
    bq = _BQ
    nvb = batch // bq
    grid = k // bq
    rows = bq // _LANES

    ids2d = ids.reshape(batch // _LANES, _LANES)
    qids2d = queue_ids.reshape(k // _LANES, _LANES)
    valid2d = valid.reshape(k // _LANES, _LANES)

    body = functools.partial(_body, nvb=nvb)

    oq, oids2d, oval2d = pl.pallas_call(
        body,
        grid=(grid,),
        in_specs=[
            pl.BlockSpec((bq, dim), lambda i: (jnp.minimum(i, nvb - 1), 0)),
            pl.BlockSpec((bq, dim), lambda i: (jnp.maximum(i, nvb), 0)),
            pl.BlockSpec((rows, _LANES), lambda i: (jnp.minimum(i, nvb - 1), 0)),
            pl.BlockSpec((rows, _LANES), lambda i: (jnp.maximum(i, nvb), 0)),
            pl.BlockSpec((rows, _LANES), lambda i: (jnp.maximum(i, nvb), 0)),
        ],
        out_specs=[
            pl.BlockSpec((bq, dim), lambda i: (i, 0)),
            pl.BlockSpec((rows, _LANES), lambda i: (i, 0)),
            pl.BlockSpec((rows, _LANES), lambda i: (i, 0)),
        ],
        out_shape=[
            jax.ShapeDtypeStruct((k, dim), queue.dtype),
            jax.ShapeDtypeStruct((k // _LANES, _LANES), queue_ids.dtype),
            jax.ShapeDtypeStruct((k // _LANES, _LANES), valid.dtype),
        ],
    )(vecs, queue, ids2d, qids2d, valid2d)

    return (oq, oids2d.reshape(k), oval2d.reshape(k))


# speedup vs baseline: 42.8890x; 1.0014x over previous
import functools

import jax
import jax.numpy as jnp
from jax.experimental import pallas as pl

_BQ = 8192


def _body(vecs_ref, queue_ref, ids_ref, qids_ref, valid_ref,
          oq_ref, oids_ref, oval_ref, *, nvb):
    i = pl.program_id(0)

    @pl.when(i < nvb)
    def _():
        oq_ref[...] = vecs_ref[...]
        oids_ref[...] = ids_ref[...]
        oval_ref[...] = jnp.ones_like(oval_ref)

    @pl.when(i >= nvb)
    def _():
        oq_ref[...] = queue_ref[...]
        oids_ref[...] = qids_ref[...]
        oval_ref[...] = valid_ref[...]


def kernel(vecs, ids, queue, queue_ids, valid):
    batch, dim = vecs.shape
    k = queue.shape[0]
    bq = _BQ
    nvb = batch // bq
    grid = k // bq

    body = functools.partial(_body, nvb=nvb)

    oq, oids, oval = pl.pallas_call(
        body,
        grid=(grid,),
        in_specs=[
            pl.BlockSpec((bq, dim), lambda i: (jnp.minimum(i, nvb - 1), 0)),
            pl.BlockSpec((bq, dim), lambda i: (jnp.maximum(i, nvb), 0)),
            pl.BlockSpec((bq,), lambda i: (jnp.minimum(i, nvb - 1),)),
            pl.BlockSpec((bq,), lambda i: (jnp.maximum(i, nvb),)),
            pl.BlockSpec((bq,), lambda i: (jnp.maximum(i, nvb),)),
        ],
        out_specs=[
            pl.BlockSpec((bq, dim), lambda i: (i, 0)),
            pl.BlockSpec((bq,), lambda i: (i,)),
            pl.BlockSpec((bq,), lambda i: (i,)),
        ],
        out_shape=[
            jax.ShapeDtypeStruct((k, dim), queue.dtype),
            jax.ShapeDtypeStruct((k,), queue_ids.dtype),
            jax.ShapeDtypeStruct((k,), valid.dtype),
        ],
    )(vecs, queue, ids, queue_ids, valid)

    return (oq, oids, oval)


# speedup vs baseline: 43.6338x; 1.0174x over previous
import jax
import jax.numpy as jnp
from jax.experimental import pallas as pl
from jax.experimental.pallas import tpu as pltpu

_LANES = 128
_R = 8192
_NBUF = 4


def _body(vecs, ids, ones, queue, qids, valid8,
          oq, oids, oval, qbuf, idbuf, vbuf, sin, sout):
    batch = vecs.shape[0]
    k = queue.shape[0]
    nc = k // _R
    nv = batch // _R
    rr = _R // _LANES

    def srcs(c):
        off = c * _R
        off2 = c * rr
        if c < nv:
            return (vecs.at[pl.ds(off, _R)], ids.at[pl.ds(off2, rr)],
                    ones.at[pl.ds(off2, rr)])
        return (queue.at[pl.ds(off, _R)], qids.at[pl.ds(off2, rr)],
                valid8.at[pl.ds(off2, rr)])

    def in_copies(c):
        b = c % _NBUF
        sq, si, sv = srcs(c)
        return (pltpu.make_async_copy(sq, qbuf.at[b], sin.at[0, b]),
                pltpu.make_async_copy(si, idbuf.at[b], sin.at[1, b]),
                pltpu.make_async_copy(sv, vbuf.at[b], sin.at[2, b]))

    def out_copies(c):
        b = c % _NBUF
        off = c * _R
        off2 = c * rr
        return (pltpu.make_async_copy(qbuf.at[b], oq.at[pl.ds(off, _R)], sout.at[0, b]),
                pltpu.make_async_copy(idbuf.at[b], oids.at[pl.ds(off2, rr)], sout.at[1, b]),
                pltpu.make_async_copy(vbuf.at[b], oval.at[pl.ds(off2, rr)], sout.at[2, b]))

    for c in range(nc):
        if c >= _NBUF:
            for cp in out_copies(c - _NBUF):
                cp.wait()
        for cp in in_copies(c):
            cp.start()
        if c >= 1:
            for cp in in_copies(c - 1):
                cp.wait()
            for cp in out_copies(c - 1):
                cp.start()
    for cp in in_copies(nc - 1):
        cp.wait()
    for cp in out_copies(nc - 1):
        cp.start()
    for c in range(nc - _NBUF, nc):
        for cp in out_copies(c):
            cp.wait()


def kernel(vecs, ids, queue, queue_ids, valid):
    batch, dim = vecs.shape
    k = queue.shape[0]
    rr = _R // _LANES
    ids2d = ids.reshape(batch // _LANES, _LANES)
    ones2d = jnp.ones((batch // _LANES, _LANES), dtype=jnp.int8)
    qids2d = queue_ids.reshape(k // _LANES, _LANES)
    valid8 = valid.astype(jnp.int8).reshape(k // _LANES, _LANES)

    hbm = pl.BlockSpec(memory_space=pltpu.MemorySpace.HBM)
    oq, oids2d, oval8 = pl.pallas_call(
        _body,
        in_specs=[hbm] * 6,
        out_specs=[hbm] * 3,
        out_shape=[
            jax.ShapeDtypeStruct((k, dim), queue.dtype),
            jax.ShapeDtypeStruct((k // _LANES, _LANES), queue_ids.dtype),
            jax.ShapeDtypeStruct((k // _LANES, _LANES), jnp.int8),
        ],
        scratch_shapes=[
            pltpu.VMEM((_NBUF, _R, dim), queue.dtype),
            pltpu.VMEM((_NBUF, rr, _LANES), queue_ids.dtype),
            pltpu.VMEM((_NBUF, rr, _LANES), jnp.int8),
            pltpu.SemaphoreType.DMA((3, _NBUF)),
            pltpu.SemaphoreType.DMA((3, _NBUF)),
        ],
    )(vecs, ids2d, ones2d, queue, qids2d, valid8)
    return (oq, oids2d.reshape(k), oval8.reshape(k).astype(jnp.bool_))
